# baseline (speedup 1.0000x reference)
import functools

import jax
import jax.numpy as jnp
from jax import lax
from jax.experimental import pallas as pl
from jax.experimental.pallas import tpu as pltpu
from jax.experimental.pallas import tpu_sc as plsc

N = 100000
E = 1600000
F = 40
H = 64
C = 3

NC, NS, L = 2, 16, 16
NW = NC * NS

ROW = 128
NROWS = 13312
E_PAD = NROWS * ROW
TRASH = N

NP = 100096
STRIPE = NP // NS

NWIN = 64
WINN = 1568
WINR = 1664
WSTRIPE = WINR // NS

NK = 4
NKI = 8
AGGW = 128


@functools.lru_cache(maxsize=None)
def _mesh():
    return plsc.VectorSubcoreMesh(
        core_axis_name="c", subcore_axis_name="s",
        num_cores=NC, num_subcores=NS)


def _sc_deg_body(dst2, z1, out, idx_b, ones_b, bounce, sdeg, sem):
    c = lax.axis_index("c")
    s = lax.axis_index("s")
    w = c * NS + s

    v1 = jnp.full((L,), 1.0, jnp.float32)

    @pl.loop(0, ROW // L)
    def _(i):
        ones_b[pl.ds(i * L, L)] = v1

    pltpu.sync_copy(z1, bounce)
    pltpu.sync_copy(bounce, sdeg.at[pl.ds(s * STRIPE, STRIPE)])
    plsc.subcore_barrier()

    rows_per_w = NROWS // NW

    @pl.loop(0, rows_per_w // NKI)
    def _(g):
        r0 = w * rows_per_w + g * NKI
        pltpu.sync_copy(dst2.at[pl.ds(r0, NKI), :], idx_b)
        for j in range(NKI):
            pltpu.sync_copy(ones_b, sdeg.at[idx_b.at[j]], add=True)

    plsc.subcore_barrier()
    pltpu.sync_copy(sdeg.at[pl.ds(s * STRIPE, STRIPE)], bounce)
    pltpu.sync_copy(bounce, out.at[pl.ds(c * NP + s * STRIPE, STRIPE)])


@functools.lru_cache(maxsize=None)
def _sc_deg():
    return pl.kernel(
        _sc_deg_body,
        out_type=jax.ShapeDtypeStruct((NC * NP,), jnp.float32),
        mesh=_mesh(),
        scratch_types=[
            pltpu.VMEM((NKI, ROW), jnp.int32),
            pltpu.VMEM((ROW,), jnp.float32),
            pltpu.VMEM((STRIPE,), jnp.float32),
            pltpu.VMEM_SHARED((NP + 8,), jnp.float32),
            pltpu.SemaphoreType.DMA,
        ],
    )


def _sc_aggwin_body(xpad, src2, dst2, zq, out, sidx, didx, rows,
                    bounce, sagg, sem, sem2):
    c = lax.axis_index("c")
    s = lax.axis_index("s")

    rows_per_t = NROWS // NS

    @pl.loop(0, NWIN // NC)
    def _(ph):
        win = c * (NWIN // NC) + ph
        base = win * WINN

        pltpu.sync_copy(zq, bounce)
        pltpu.sync_copy(bounce, sagg.at[pl.ds(s * WSTRIPE, WSTRIPE), :])
        plsc.subcore_barrier()

        @pl.loop(0, rows_per_t // NKI)
        def _(g):
            r0 = s * rows_per_t + g * NKI
            pltpu.sync_copy(src2.at[pl.ds(r0, NKI), :], sidx)
            pltpu.sync_copy(dst2.at[pl.ds(r0, NKI), :], didx)
            for jj in range(NKI):
                for grp in range(ROW // L):
                    v = didx[jj, pl.ds(grp * L, L)] - base
                    keep = v.astype(jnp.uint32) < jnp.uint32(WINN)
                    didx[jj, pl.ds(grp * L, L)] = jnp.where(
                        keep, v, jnp.int32(WINN))
                    sv = sidx[jj, pl.ds(grp * L, L)]
                    sidx[jj, pl.ds(grp * L, L)] = jnp.where(
                        keep, sv, jnp.int32(0))
            for q in range(NKI // NK):
                descs = [
                    pltpu.async_copy(xpad.at[sidx.at[q * NK + j]],
                                     rows.at[j], sem)
                    for j in range(NK)]
                scds = []
                for j in range(NK):
                    descs[j].wait()
                    scds.append(pltpu.async_copy(
                        rows.at[j], sagg.at[didx.at[q * NK + j]], sem2,
                        add=True))
                for d in scds:
                    d.wait()

        plsc.subcore_barrier()
        pltpu.sync_copy(sagg.at[pl.ds(s * WSTRIPE, WSTRIPE), :], bounce)
        pltpu.sync_copy(
            bounce, out.at[pl.ds(win * WINR + s * WSTRIPE, WSTRIPE), :])
        plsc.subcore_barrier()


@functools.lru_cache(maxsize=None)
def _sc_aggwin():
    return pl.kernel(
        _sc_aggwin_body,
        out_type=jax.ShapeDtypeStruct((NWIN * WINR, AGGW), jnp.float32),
        mesh=_mesh(),
        scratch_types=[
            pltpu.VMEM((NKI, ROW), jnp.int32),
            pltpu.VMEM((NKI, ROW), jnp.int32),
            pltpu.VMEM((NK, ROW, ROW), jnp.float32),
            pltpu.VMEM((WSTRIPE, AGGW), jnp.float32),
            pltpu.VMEM_SHARED((WINR, AGGW), jnp.float32),
            pltpu.SemaphoreType.DMA,
            pltpu.SemaphoreType.DMA,
        ],
    )


def _unwindow(aggflat):
    return aggflat.reshape(NWIN, WINR, AGGW)[:, :WINN, :].reshape(
        NWIN * WINN, AGGW)


BT = 4000
GRID = N // BT


def _tc_a_body(x_ref, dis_ref, xp_ref):
    xs = x_ref[...] * dis_ref[...]
    xp_ref[...] = jnp.concatenate(
        [xs, jnp.zeros((BT, ROW - F), jnp.float32)], axis=1)


def _tc_a(x, ddis):
    return pl.pallas_call(
        _tc_a_body,
        grid=(GRID,),
        in_specs=[
            pl.BlockSpec((BT, F), lambda i: (i, 0)),
            pl.BlockSpec((BT, 1), lambda i: (i, 0)),
        ],
        out_specs=pl.BlockSpec((BT, ROW), lambda i: (i, 0)),
        out_shape=jax.ShapeDtypeStruct((N, ROW), jnp.float32),
    )(x, ddis)


def _tc_b_body(agg_ref, dis_ref, w1_ref, b1_ref, ws_ref, bs_ref,
               w2_ref, lp_ref):
    dis = dis_ref[...]
    a = agg_ref[:, :F] * dis
    h = lax.dot_general(a, w1_ref[...], (((1,), (1,)), ((), ())),
                        preferred_element_type=jnp.float32) + b1_ref[...]
    h = jnp.maximum(h, 0.0)
    mem = lax.dot_general(h, ws_ref[...], (((1,), (1,)), ((), ())),
                          preferred_element_type=jnp.float32) + bs_ref[...]
    spike = jnp.where(mem >= 0.5, 1.0, 0.0)
    logits = lax.dot_general(spike, w2_ref[...], (((1,), (1,)), ((), ())),
                             preferred_element_type=jnp.float32)
    lp_ref[...] = jnp.concatenate(
        [logits * dis, jnp.zeros((BT, ROW - C), jnp.float32)], axis=1)


def _tc_b(agg1, ddis, W1, b1r, Ws, bsr, W2):
    return pl.pallas_call(
        _tc_b_body,
        grid=(GRID,),
        in_specs=[
            pl.BlockSpec((BT, AGGW), lambda i: (i, 0)),
            pl.BlockSpec((BT, 1), lambda i: (i, 0)),
            pl.BlockSpec((H, F), lambda i: (0, 0)),
            pl.BlockSpec((1, H), lambda i: (0, 0)),
            pl.BlockSpec((H, H), lambda i: (0, 0)),
            pl.BlockSpec((1, H), lambda i: (0, 0)),
            pl.BlockSpec((C, H), lambda i: (0, 0)),
        ],
        out_specs=pl.BlockSpec((BT, ROW), lambda i: (i, 0)),
        out_shape=jax.ShapeDtypeStruct((N, ROW), jnp.float32),
    )(agg1, ddis, W1, b1r, Ws, bsr, W2)


def _tc_c_body(agg_ref, dis_ref, b2_ref, ef_ref, o_ref):
    acc = agg_ref[:, :C] * dis_ref[...]
    o_ref[...] = (acc + b2_ref[...]) * ef_ref[...]


def _tc_c(agg2, ddis, b2r, efr):
    return pl.pallas_call(
        _tc_c_body,
        grid=(GRID,),
        in_specs=[
            pl.BlockSpec((BT, AGGW), lambda i: (i, 0)),
            pl.BlockSpec((BT, 1), lambda i: (i, 0)),
            pl.BlockSpec((1, C), lambda i: (0, 0)),
            pl.BlockSpec((1, C), lambda i: (0, 0)),
        ],
        out_specs=pl.BlockSpec((BT, C), lambda i: (i, 0)),
        out_shape=jax.ShapeDtypeStruct((N, C), jnp.float32),
    )(agg2, ddis, b2r, efr)


@jax.jit
def kernel(x, edge_index, W1, b1, Ws, bs, W2, b2, ethical_filter):
    src = edge_index[0].astype(jnp.int32)
    dst = edge_index[1].astype(jnp.int32)
    loop = jnp.arange(N, dtype=jnp.int32)
    npad = E_PAD - E - N
    src_p = jnp.concatenate([src, loop, jnp.zeros((npad,), jnp.int32)])
    dst_p = jnp.concatenate([dst, loop, jnp.full((npad,), TRASH, jnp.int32)])
    dst2 = dst_p.reshape(NROWS, ROW)
    src2 = src_p.reshape(NROWS, ROW)

    degf = _sc_deg()(dst2, jnp.zeros((STRIPE,), jnp.float32))
    deg = degf[:N] + degf[NP:NP + N]
    ddis = (deg ** -0.5).reshape(N, 1)

    xpad = _tc_a(x, ddis)

    zq = jnp.zeros((WSTRIPE, AGGW), jnp.float32)
    agg1 = _unwindow(_sc_aggwin()(xpad, src2, dst2, zq))

    lp = _tc_b(agg1, ddis, W1, b1.reshape(1, H), Ws,
               bs.reshape(1, H), W2)

    agg2 = _unwindow(_sc_aggwin()(lp, src2, dst2, zq))

    return _tc_c(agg2, ddis, b2.reshape(1, C),
                 ethical_filter.reshape(1, C))
